# baseline (speedup 1.0000x reference)
import jax
import jax.numpy as jnp
from jax.experimental import pallas as pl
from jax.experimental.pallas import tpu as pltpu

N, E = 1024, 2048
NFEAT_V, NFEAT_E, NHID, NCLASS = 128, 16, 64, 16
BN = 256
BE = 256
PH1, PH2 = 4, 12


def _bf(x):
    return x.astype(jnp.bfloat16).astype(jnp.float32)


def _cmr(rows, cols):
    return (jax.lax.broadcasted_iota(jnp.int32, (rows, cols), 1)
            - jax.lax.broadcasted_iota(jnp.int32, (rows, cols), 0))


def _fused_kernel(T_ref, Tc_ref, adj_v_ref, adj_e_ref, X_ref, Z_ref,
                  W1_ref, p1_ref, b1_ref, W2_ref, p2_ref, b2_ref,
                  W3_ref, p3_ref, b3_ref, out_ref,
                  Xh, Zh, Tbf, Td, HWbf, ZWbf, d1, d2, d3, cmr_n, cmr_e):
    s = pl.program_id(0)

    @pl.when(s == 0)
    def _init1():
        d1v = jnp.sum(_bf(Z_ref[...]) * _bf(p1_ref[...]), axis=1)
        d1[...] = d1v.reshape(1, E)
        Tbf[...] = T_ref[...].astype(jnp.bfloat16)
        Td[...] = (T_ref[...] * d1v.reshape(1, E)).astype(jnp.bfloat16)
        HWbf[...] = jnp.dot(X_ref[...], W1_ref[...],
                            preferred_element_type=jnp.float32
                            ).astype(jnp.bfloat16)
        cmr_n[...] = _cmr(BN, N)
        cmr_e[...] = _cmr(BE, E)

    @pl.when(s < PH1)
    def _gc1():
        i = s
        mult = jax.lax.dot_general(
            Td[pl.ds(i * BN, BN), :], Tbf[...], (((1,), (1,)), ((), ())),
            preferred_element_type=jnp.float32)
        A = jnp.where(cmr_n[...] == i * BN, 1.0, mult) \
            * adj_v_ref[pl.ds(i * BN, BN), :]
        out = jnp.dot(A.astype(jnp.bfloat16), HWbf[...],
                      preferred_element_type=jnp.float32) + b1_ref[...]
        Xh[pl.ds(i * BN, BN), :] = jnp.maximum(out, 0.0)

    @pl.when(s == PH1)
    def _init2():
        d2[...] = jnp.sum(_bf(Xh[...]) * _bf(p2_ref[...]), axis=1,
                          keepdims=True)
        ZWbf[...] = jnp.dot(jnp.maximum(Z_ref[...], 0.0), W2_ref[...],
                            preferred_element_type=jnp.float32
                            ).astype(jnp.bfloat16)

    @pl.when((s >= PH1) & (s < PH2))
    def _gc2():
        j = s - PH1
        mult = jax.lax.dot_general(
            (Tc_ref[...] * d2[...]).astype(jnp.bfloat16), Tbf[...],
            (((0,), (0,)), ((), ())), preferred_element_type=jnp.float32)
        A = jnp.where(cmr_e[...] == j * BE, 1.0, mult) * adj_e_ref[...]
        out = jnp.dot(A.astype(jnp.bfloat16), ZWbf[...],
                      preferred_element_type=jnp.float32) + b2_ref[...]
        Zh[pl.ds(j * BE, BE), :] = jnp.maximum(out, 0.0)

    @pl.when(s == PH2)
    def _init3():
        d3v = jnp.sum(_bf(Zh[...]) * _bf(p3_ref[...]), axis=1)
        d3[...] = d3v.reshape(1, E)
        Td[...] = (T_ref[...] * d3v.reshape(1, E)).astype(jnp.bfloat16)
        HWbf[:, :NCLASS] = jnp.dot(Xh[...], W3_ref[...],
                                   preferred_element_type=jnp.float32
                                   ).astype(jnp.bfloat16)

    @pl.when(s >= PH2)
    def _gc3():
        i = s - PH2
        mult = jax.lax.dot_general(
            Td[pl.ds(i * BN, BN), :], Tbf[...], (((1,), (1,)), ((), ())),
            preferred_element_type=jnp.float32)
        A = jnp.where(cmr_n[...] == i * BN, 1.0, mult) \
            * adj_v_ref[pl.ds(i * BN, BN), :]
        out = jnp.dot(A.astype(jnp.bfloat16), HWbf[:, :NCLASS],
                      preferred_element_type=jnp.float32) + b3_ref[...]
        shifted = out - jnp.max(out, axis=1, keepdims=True)
        out_ref[...] = shifted - jnp.log(jnp.sum(jnp.exp(shifted), axis=1,
                                                 keepdims=True))


def kernel(X, Z, adj_e, adj_v, T, W1, p1, b1, W2, p2, b2, W3, p3, b3):
    b1r, b2r, b3r = b1.reshape(1, -1), b2.reshape(1, -1), b3.reshape(1, -1)
    const = lambda a, b: (lambda s: (a, b))
    return pl.pallas_call(
        _fused_kernel,
        grid=(16,),
        in_specs=[
            pl.BlockSpec((N, E), const(0, 0)),
            pl.BlockSpec((N, BE), lambda s: (0, jnp.clip(s - PH1, 0, 7))),
            pl.BlockSpec((N, N), const(0, 0)),
            pl.BlockSpec((BE, E), lambda s: (jnp.clip(s - PH1, 0, 7), 0)),
            pl.BlockSpec((N, NFEAT_V), const(0, 0)),
            pl.BlockSpec((E, NFEAT_E), const(0, 0)),
            pl.BlockSpec((NFEAT_V, NHID), const(0, 0)),
            pl.BlockSpec((1, NFEAT_E), const(0, 0)),
            pl.BlockSpec((1, NHID), const(0, 0)),
            pl.BlockSpec((NFEAT_E, NFEAT_E), const(0, 0)),
            pl.BlockSpec((1, NHID), const(0, 0)),
            pl.BlockSpec((1, NFEAT_E), const(0, 0)),
            pl.BlockSpec((NHID, NCLASS), const(0, 0)),
            pl.BlockSpec((1, NFEAT_E), const(0, 0)),
            pl.BlockSpec((1, NCLASS), const(0, 0)),
        ],
        out_specs=pl.BlockSpec((BN, NCLASS),
                               lambda s: (jnp.clip(s - PH2, 0, 3), 0)),
        out_shape=jax.ShapeDtypeStruct((N, NCLASS), jnp.float32),
        scratch_shapes=[
            pltpu.VMEM((N, NHID), jnp.float32),
            pltpu.VMEM((E, NFEAT_E), jnp.float32),
            pltpu.VMEM((N, E), jnp.bfloat16),
            pltpu.VMEM((N, E), jnp.bfloat16),
            pltpu.VMEM((N, NHID), jnp.bfloat16),
            pltpu.VMEM((E, NFEAT_E), jnp.bfloat16),
            pltpu.VMEM((1, E), jnp.float32),
            pltpu.VMEM((N, 1), jnp.float32),
            pltpu.VMEM((1, E), jnp.float32),
            pltpu.VMEM((BN, N), jnp.int32),
            pltpu.VMEM((BE, E), jnp.int32),
        ],
    )(T, T, adj_v, adj_e, X, Z, W1, p1, b1r, W2, p2, b2r, W3, p3, b3r)
